# baseline (speedup 1.0000x reference)
import functools

import jax
import jax.numpy as jnp
from jax import lax
from jax.experimental import pallas as pl
from jax.experimental.pallas import tpu as pltpu
from jax.experimental.pallas import tpu_sc as plsc

_NB = 5
_L = 16
_NC = 2
_NS = 16
_NW = _NC * _NS
_C2_BIN2 = 1.8944272
_C2_OUT = 0.10557281

_GDN = lax.GatherDimensionNumbers(
    offset_dims=(), collapsed_slice_dims=(0,), start_index_map=(0,))


def _rot(v, idx):
    return lax.gather(v, idx, _GDN, (1,),
                      mode=lax.GatherScatterMode.PROMISE_IN_BOUNDS)


def _fpfh_body(inp_hbm, out_hbm, slab, hist, hist2, result):
    Bz = inp_hbm.shape[0]
    Pz = inp_hbm.shape[2]
    ext = Pz + Pz // 2
    per_batch = _NW // Bz
    ngrp = Pz // _L
    gpw = ngrp // per_batch
    nt = (Pz // 2) // _L
    wid = lax.axis_index("s") * _NC + lax.axis_index("c")
    b = wid // per_batch
    q = wid % per_batch

    pltpu.sync_copy(inp_hbm.at[b], slab.at[:, pl.ds(0, Pz)])

    zeros16 = jnp.zeros((_L,), jnp.float32)

    def wrap_body(c, acc):
        cs = c * _L
        for r in range(6):
            slab[r, pl.ds(Pz + cs, _L)] = slab[r, pl.ds(cs, _L)]
        return acc

    lax.fori_loop(0, (ext - Pz) // _L, wrap_body, 0)

    def zero_body(k, c):
        hist[pl.ds(k * _L, _L)] = zeros16
        hist2[pl.ds(k * _L, _L)] = zeros16
        return c

    lax.fori_loop(0, 128, zero_body, 0)

    lane = lax.iota(jnp.int32, _L)
    base992 = lane.astype(jnp.float32) + float(62 * _L)
    ones16 = jnp.ones((_L,), jnp.float32)
    half16 = jnp.full((_L,), 0.5, jnp.float32)
    ridx = [((lane + rot) & (_L - 1)).reshape(_L, 1) for rot in range(_L)]

    def pair_body(pix, piy, piz, nix, niy, niz, nni,
                  pjx, pjy, pjz, njx, njy, njz, nnj, val):
        dx = pjx - pix
        dy = pjy - piy
        dz = pjz - piz
        r2 = dx * dx + dy * dy + dz * dz
        d = nix * dx + niy * dy + niz * dz
        dn = njx * dx + njy * dy + njz * dz
        u = nix * njx + niy * njy + niz * njz
        vx = dy * niz - dz * niy
        vy = dz * nix - dx * niz
        vz = dx * niy - dy * nix
        m = vx * vx + vy * vy + vz * vz
        s = vx * njx + vy * njy + vz * njz
        yv = dn * nni - d * u
        s2 = s * s
        d2 = d * d
        y2 = yv * yv
        uu = u * u
        sa = jnp.where(s >= 0, 1.0, -1.0)
        tpa = 0.36 * r2
        tpb = 0.04 * r2
        ugt = u > 0
        ult = u < 0

        ca = (jnp.where(s2 > 0.36 * m, 400.0, 0.0)
              + jnp.where(s2 > 0.04 * m, 400.0, 0.0))
        cp = (jnp.where(d2 > tpa, 80.0, 0.0)
              + jnp.where(d2 > tpb, 80.0, 0.0))
        sp = jnp.where(d >= 0, 1.0, -1.0)
        x2 = uu * nni * m
        is2 = ugt & (x2 >= _C2_BIN2 * y2)
        iso = ult & (x2 > _C2_OUT * y2)
        sy = jnp.where(yv >= 0, 16.0, -16.0)
        dt = jnp.where(is2, 0.0, sy + jnp.where(iso, sy, 0.0))
        addr = (base992 + sa * ca + sp * cp + dt).astype(jnp.int32)
        plsc.addupdate_scatter(hist, [addr], val)

        ddn = dn * dn
        mr = r2 * nnj - ddn
        yr = dn * u - d * nnj
        y2r = yr * yr
        car = (jnp.where(s2 > 0.36 * mr, 400.0, 0.0)
               + jnp.where(s2 > 0.04 * mr, 400.0, 0.0))
        cpr = (jnp.where(ddn > tpa, 80.0, 0.0)
               + jnp.where(ddn > tpb, 80.0, 0.0))
        spr = jnp.where(dn <= 0, 1.0, -1.0)
        x2r = uu * nnj * mr
        is2r = ugt & (x2r >= _C2_BIN2 * y2r)
        isor = ult & (x2r > _C2_OUT * y2r)
        syr = jnp.where(yr >= 0, 16.0, -16.0)
        dtr = jnp.where(is2r, 0.0, syr + jnp.where(isor, syr, 0.0))
        addr_r = (base992 + sa * car + spr * cpr + dtr).astype(jnp.int32)
        plsc.addupdate_scatter(hist2, [addr_r], val)

    def g_body(gi, c0):
        gb = (q * gpw + gi) * _L
        pix = slab[0, pl.ds(gb, _L)]
        piy = slab[1, pl.ds(gb, _L)]
        piz = slab[2, pl.ds(gb, _L)]
        nix = slab[3, pl.ds(gb, _L)]
        niy = slab[4, pl.ds(gb, _L)]
        niz = slab[5, pl.ds(gb, _L)]
        nni = nix * nix + niy * niy + niz * niz

        for rot in range(1, _L):
            ix = ridx[rot]
            pair_body(pix, piy, piz, nix, niy, niz, nni,
                      _rot(pix, ix), _rot(piy, ix), _rot(piz, ix),
                      _rot(nix, ix), _rot(niy, ix), _rot(niz, ix),
                      _rot(nni, ix), half16)

        def t_body(t, c1):
            js = gb + t * _L
            qjx = slab[0, pl.ds(js, _L)]
            qjy = slab[1, pl.ds(js, _L)]
            qjz = slab[2, pl.ds(js, _L)]
            wjx = slab[3, pl.ds(js, _L)]
            wjy = slab[4, pl.ds(js, _L)]
            wjz = slab[5, pl.ds(js, _L)]
            nnq = wjx * wjx + wjy * wjy + wjz * wjz
            val = jnp.where(t == nt, half16, ones16)
            pair_body(pix, piy, piz, nix, niy, niz, nni,
                      qjx, qjy, qjz, wjx, wjy, wjz, nnq, val)
            for rot in range(1, _L):
                ix = ridx[rot]
                pair_body(pix, piy, piz, nix, niy, niz, nni,
                          _rot(qjx, ix), _rot(qjy, ix), _rot(qjz, ix),
                          _rot(wjx, ix), _rot(wjy, ix), _rot(wjz, ix),
                          _rot(nnq, ix), val)
            return c1

        lax.fori_loop(1, nt + 1, t_body, c0)
        return c0

    lax.fori_loop(0, gpw, g_body, 0)

    def red_body(cch, c):
        addr0 = lane * _L + cch * (_L * _L)
        acc = zeros16
        for l in range(_L):
            acc = acc + (plsc.load_gather(hist, [addr0 + l])
                         + plsc.load_gather(hist2, [addr0 + l]))
        result[pl.ds(cch * _L, _L)] = acc
        return c

    lax.fori_loop(0, 128 // _L, red_body, 0)
    pltpu.sync_copy(result, out_hbm.at[wid])


def kernel(input):
    Bz, Pz, _ = input.shape
    inp_t = jnp.transpose(input, (0, 2, 1))
    ext = Pz + Pz // 2

    mesh = plsc.VectorSubcoreMesh(
        core_axis_name="c", subcore_axis_name="s",
        num_cores=_NC, num_subcores=_NS)
    run = functools.partial(
        pl.kernel,
        out_type=jax.ShapeDtypeStruct((_NW, 128), jnp.float32),
        mesh=mesh,
        scratch_types=[
            pltpu.VMEM((6, ext), jnp.float32),
            pltpu.VMEM((128 * _L,), jnp.float32),
            pltpu.VMEM((128 * _L,), jnp.float32),
            pltpu.VMEM((128,), jnp.float32),
        ],
        compiler_params=pltpu.CompilerParams(needs_layout_passes=False),
    )(_fpfh_body)
    part = run(inp_t)

    per_batch = _NW // Bz
    hist = part.reshape(Bz, per_batch, 128).sum(axis=1)[:, : _NB ** 3]
    return hist / float(Pz * (Pz - 1))


# speedup vs baseline: 1.5544x; 1.5544x over previous
import functools

import jax
import jax.numpy as jnp
from jax import lax
from jax.experimental import pallas as pl
from jax.experimental.pallas import tpu as pltpu
from jax.experimental.pallas import tpu_sc as plsc

_NB = 5
_L = 16
_NC = 2
_NS = 16
_NW = _NC * _NS
_C2_BIN2 = 1.8944272
_C2_OUT = 0.10557281


def _fpfh_body(inp_hbm, out_hbm, slab, hist, result):
    Bz = inp_hbm.shape[0]
    Pz = inp_hbm.shape[2]
    wid = lax.axis_index("s") * _NC + lax.axis_index("c")
    per_batch = _NW // Bz
    half = Pz // (2 * per_batch)
    grp = half // _L
    nchunk = Pz // _L
    b = wid // per_batch
    q = wid % per_batch

    pltpu.sync_copy(inp_hbm.at[b], slab)

    zeros16 = jnp.zeros((_L,), jnp.float32)

    def zero_body(k, c):
        hist[pl.ds(k * _L, _L)] = zeros16
        return c

    lax.fori_loop(0, 128, zero_body, 0)

    lane = lax.iota(jnp.int32, _L)
    base992 = lane.astype(jnp.float32) + float(62 * _L)
    ones16 = jnp.ones((_L,), jnp.float32)

    for blk in range(2):
        base_blk = q * half if blk == 0 else (Pz - half) - q * half

        def g_body(g, c0, base_blk=base_blk):
            gb = base_blk + g * _L
            jc0 = gb // _L
            pivx = slab[0, pl.ds(gb, _L)]
            pivy = slab[1, pl.ds(gb, _L)]
            pivz = slab[2, pl.ds(gb, _L)]
            nivx = slab[3, pl.ds(gb, _L)]
            nivy = slab[4, pl.ds(gb, _L)]
            nivz = slab[5, pl.ds(gb, _L)]

            def jc_body(jc, c1):
                js = jc * _L
                pjx = slab[0, pl.ds(js, _L)]
                pjy = slab[1, pl.ds(js, _L)]
                pjz = slab[2, pl.ds(js, _L)]
                njx = slab[3, pl.ds(js, _L)]
                njy = slab[4, pl.ds(js, _L)]
                njz = slab[5, pl.ds(js, _L)]
                nnj = njx * njx + njy * njy + njz * njz
                jglob = lane + js
                for k in range(_L):
                    pix = pivx[k]
                    piy = pivy[k]
                    piz = pivz[k]
                    nix = nivx[k]
                    niy = nivy[k]
                    niz = nivz[k]
                    nn = nix * nix + niy * niy + niz * niz
                    mask = jglob > (gb + k)

                    dx = pjx - pix
                    dy = pjy - piy
                    dz = pjz - piz
                    r2 = dx * dx + dy * dy + dz * dz
                    d = nix * dx + niy * dy + niz * dz
                    dn = njx * dx + njy * dy + njz * dz
                    u = nix * njx + niy * njy + niz * njz
                    vx = dy * niz - dz * niy
                    vy = dz * nix - dx * niz
                    vz = dx * niy - dy * nix
                    m = vx * vx + vy * vy + vz * vz
                    s = vx * njx + vy * njy + vz * njz
                    yv = dn * nn - d * u
                    s2 = s * s
                    d2 = d * d
                    y2 = yv * yv
                    uu = u * u
                    sa = jnp.where(s >= 0, 1.0, -1.0)
                    tpa = 0.36 * r2
                    tpb = 0.04 * r2

                    ca = (jnp.where(s2 > 0.36 * m, 400.0, 0.0)
                          + jnp.where(s2 > 0.04 * m, 400.0, 0.0))
                    cp = (jnp.where(d2 > tpa, 80.0, 0.0)
                          + jnp.where(d2 > tpb, 80.0, 0.0))
                    sp = jnp.where(d >= 0, 1.0, -1.0)
                    x2 = uu * nn * m
                    is2 = (u > 0) & (x2 >= _C2_BIN2 * y2)
                    iso = (u < 0) & (x2 > _C2_OUT * y2)
                    sy = jnp.where(yv >= 0, 16.0, -16.0)
                    dt = jnp.where(is2, 0.0, sy + jnp.where(iso, sy, 0.0))
                    addr = (base992 + sa * ca + sp * cp + dt).astype(jnp.int32)
                    plsc.addupdate_scatter(hist, [addr], ones16, mask=mask)

                    ddn = dn * dn
                    mr = r2 * nnj - ddn
                    yr = dn * u - d * nnj
                    y2r = yr * yr
                    car = (jnp.where(s2 > 0.36 * mr, 400.0, 0.0)
                           + jnp.where(s2 > 0.04 * mr, 400.0, 0.0))
                    cpr = (jnp.where(ddn > tpa, 80.0, 0.0)
                           + jnp.where(ddn > tpb, 80.0, 0.0))
                    spr = jnp.where(dn <= 0, 1.0, -1.0)
                    x2r = uu * nnj * mr
                    is2r = (u > 0) & (x2r >= _C2_BIN2 * y2r)
                    isor = (u < 0) & (x2r > _C2_OUT * y2r)
                    syr = jnp.where(yr >= 0, 16.0, -16.0)
                    dtr = jnp.where(is2r, 0.0, syr + jnp.where(isor, syr, 0.0))
                    addr_r = (base992 + sa * car + spr * cpr + dtr).astype(jnp.int32)
                    plsc.addupdate_scatter(hist, [addr_r], ones16, mask=mask)
                return c1

            lax.fori_loop(jc0, nchunk, jc_body, c0)
            return c0

        lax.fori_loop(0, grp, g_body, 0)

    def red_body(cch, c):
        addr0 = lane * _L + cch * (_L * _L)
        acc = zeros16
        for l in range(_L):
            acc = acc + plsc.load_gather(hist, [addr0 + l])
        result[pl.ds(cch * _L, _L)] = acc
        return c

    lax.fori_loop(0, 128 // _L, red_body, 0)
    pltpu.sync_copy(result, out_hbm.at[wid])


def kernel(input):
    Bz, Pz, _ = input.shape
    inp_t = jnp.transpose(input, (0, 2, 1))

    mesh = plsc.VectorSubcoreMesh(
        core_axis_name="c", subcore_axis_name="s",
        num_cores=_NC, num_subcores=_NS)
    run = functools.partial(
        pl.kernel,
        out_type=jax.ShapeDtypeStruct((_NW, 128), jnp.float32),
        mesh=mesh,
        scratch_types=[
            pltpu.VMEM((6, Pz), jnp.float32),
            pltpu.VMEM((128 * _L,), jnp.float32),
            pltpu.VMEM((128,), jnp.float32),
        ],
        compiler_params=pltpu.CompilerParams(needs_layout_passes=False),
    )(_fpfh_body)
    part = run(inp_t)

    per_batch = _NW // Bz
    hist = part.reshape(Bz, per_batch, 128).sum(axis=1)[:, : _NB ** 3]
    return hist / float(Pz * (Pz - 1))
